# baseline (speedup 1.0000x reference)
import functools

import jax
import jax.numpy as jnp
from jax import lax
from jax.experimental import pallas as pl
from jax.experimental.pallas import tpu as pltpu
from jax.experimental.pallas import tpu_sc as plsc

N = 10000
E = 320000
D = 128
EA = 16

NC, NS, L = 2, 16, 16
NW = NC * NS
NPART = 2
EP = E // NPART
EPW = EP // NW
CHUNK = 40
NCH = EPW // CHUNK
ROWS_PT = 624
ROWS_TAIL = N - NS * ROWS_PT

_MESH = plsc.VectorSubcoreMesh(core_axis_name="c", subcore_axis_name="s")


def _rnd_bf16_bits(x):
    u = lax.bitcast_convert_type(x, jnp.uint32)
    return (u + jnp.uint32(0x7FFF) + ((u >> 16) & jnp.uint32(1))) >> 16


def _pq_body(h_ref, wa_ref, wb_ref, t_ref):
    hb = h_ref[...]
    pf = jnp.dot(hb, wa_ref[...], preferred_element_type=jnp.float32)
    qf = jnp.dot(hb, wb_ref[...], preferred_element_type=jnp.float32)
    packed = _rnd_bf16_bits(pf) | (_rnd_bf16_bits(qf) << 16)
    t_ref[...] = lax.bitcast_convert_type(packed, jnp.int32)


def _compute_pq(h, wa, wb):
    nb = 10
    blk = N // nb
    return pl.pallas_call(
        _pq_body,
        grid=(nb,),
        in_specs=[
            pl.BlockSpec((blk, D), lambda i: (i, 0)),
            pl.BlockSpec((D, D), lambda i: (0, 0)),
            pl.BlockSpec((D, D), lambda i: (0, 0)),
        ],
        out_specs=pl.BlockSpec((blk, D), lambda i: (i, 0)),
        out_shape=jax.ShapeDtypeStruct((N, D), jnp.int32),
    )(h, wa, wb)


NBUF = 4
GNBUF = 2
assert NCH % NBUF == 1 and NCH % GNBUF == 1


@functools.partial(
    pl.kernel,
    out_type=jax.ShapeDtypeStruct((EP, D // 2), jnp.int32),
    mesh=_MESH,
    scratch_types=[
        pltpu.VMEM_SHARED((N, D), jnp.int32),
        pltpu.VMEM((4, CHUNK), jnp.int32),
        pltpu.VMEM((4, CHUNK), jnp.int32),
        pltpu.VMEM((GNBUF, CHUNK, D), jnp.int32),
        pltpu.VMEM((GNBUF, CHUNK, D), jnp.int32),
        pltpu.VMEM((GNBUF, CHUNK, D // 2), jnp.int32),
    ] + [pltpu.SemaphoreType.DMA] * (3 * GNBUF + 8),
)
def _gather_add(t_hbm, src_hbm, dst_hbm, g_hbm,
                spt, sidx, didx, pbuf, qbuf, gbuf, *sems):
    c = lax.axis_index("c")
    s = lax.axis_index("s")
    wid = s * NC + c
    base = wid * EPW

    pltpu.sync_copy(t_hbm.at[pl.ds(s * ROWS_PT, ROWS_PT)],
                    spt.at[pl.ds(s * ROWS_PT, ROWS_PT)])

    @pl.when(s == NS - 1)
    def _stage_tail():
        pltpu.sync_copy(t_hbm.at[pl.ds(NS * ROWS_PT, ROWS_TAIL)],
                        spt.at[pl.ds(NS * ROWS_PT, ROWS_TAIL)])

    plsc.subcore_barrier()

    semp = sems[0:GNBUF]
    semq = sems[GNBUF:2 * GNBUF]
    semw = sems[2 * GNBUF:3 * GNBUF]
    semi = sems[3 * GNBUF:3 * GNBUF + 4]
    semj = sems[3 * GNBUF + 4:3 * GNBUF + 8]

    def startidx(j, i4):
        pltpu.async_copy(src_hbm.at[wid, j], sidx.at[i4], semi[i4])
        pltpu.async_copy(dst_hbm.at[wid, j], didx.at[i4], semj[i4])

    def waitidx(j, i4):
        pltpu.make_async_copy(src_hbm.at[wid, j], sidx.at[i4],
                              semi[i4]).wait()
        pltpu.make_async_copy(dst_hbm.at[wid, j], didx.at[i4],
                              semj[i4]).wait()

    def startg(b, i4):
        pltpu.async_copy(spt.at[sidx.at[i4]], pbuf.at[b], semp[b])
        pltpu.async_copy(spt.at[didx.at[i4]], qbuf.at[b], semq[b])

    def waitg(b, i4):
        pltpu.make_async_copy(spt.at[sidx.at[i4]], pbuf.at[b],
                              semp[b]).wait()
        pltpu.make_async_copy(spt.at[didx.at[i4]], qbuf.at[b],
                              semq[b]).wait()

    def startw(j, b):
        pltpu.async_copy(gbuf.at[b],
                         g_hbm.at[pl.ds(base + j * CHUNK, CHUNK)], semw[b])

    def waitw(b):
        pltpu.make_async_copy(gbuf.at[b],
                              g_hbm.at[pl.ds(base, CHUNK)], semw[b]).wait()

    kone = jnp.full((L,), 1, jnp.int32)
    khalf = jnp.full((L,), 0x7FFF, jnp.int32)
    kmhi = jnp.full((L,), -65536, jnp.int32)
    k16 = jnp.full((L,), 16, jnp.int32)

    def _to_bf16_bits(sf):
        u = lax.bitcast_convert_type(sf, jnp.int32)
        r = u + khalf + (lax.shift_right_logical(u, k16) & kone)
        return lax.shift_right_logical(r, k16)

    def adds(j, b):
        @pl.loop(0, CHUNK)
        def _row(r):
            for v in range(D // 2 // L):
                lo = pl.ds(v * L, L)
                hi = pl.ds(D // 2 + v * L, L)
                p_lo = lax.bitcast_convert_type(
                    lax.shift_left(pbuf[b, r, lo], k16), jnp.float32)
                q_lo = lax.bitcast_convert_type(
                    qbuf[b, r, lo] & kmhi, jnp.float32)
                p_hi = lax.bitcast_convert_type(
                    lax.shift_left(pbuf[b, r, hi], k16), jnp.float32)
                q_hi = lax.bitcast_convert_type(
                    qbuf[b, r, hi] & kmhi, jnp.float32)
                r_lo = _to_bf16_bits(p_lo + q_lo)
                r_hi = _to_bf16_bits(p_hi + q_hi)
                gbuf[b, r, lo] = r_lo | lax.shift_left(r_hi, k16)

    for k in range(4):
        startidx(k, k)
    waitidx(0, 0)
    startg(0, 0)
    waitidx(1, 1)
    startg(1, 1)

    assert (NCH - 1) % 4 == 0

    @pl.loop(0, NCH - 1, step=4)
    def _grp(j):
        for k in range(4):
            c = j + k
            b = k % 2
            waitg(b, k)

            @pl.when(c >= 2)
            def _ww(b=b):
                waitw(b)

            adds(c, b)
            startw(c, b)
            cn = c + 2

            @pl.when(cn < NCH)
            def _pf(cn=cn, b=b, i4=(k + 2) % 4):
                waitidx(cn, i4)
                startg(b, i4)

            ci = c + 4

            @pl.when(ci < NCH)
            def _pfi(ci=ci, i4=k):
                startidx(ci, i4)

    waitg(0, 0)
    waitw(0)
    adds(NCH - 1, 0)
    startw(NCH - 1, 0)
    for b in range(GNBUF):
        waitw(b)


def _unpack_bf16(w):
    u = lax.bitcast_convert_type(w, jnp.uint32)
    lo = lax.bitcast_convert_type(u << 16, jnp.float32)
    hi = lax.bitcast_convert_type(u & jnp.uint32(0xFFFF0000), jnp.float32)
    return jnp.concatenate([lo, hi], axis=1)


def _msg_body(g_ref, ea_ref, wc_ref, b1_ref, w2_ref, b2_ref, w3_ref, b3_ref,
              gg_ref, bb_ref, out_ref):
    x = _unpack_bf16(g_ref[...]) + jnp.dot(
        ea_ref[...], wc_ref[...],
        preferred_element_type=jnp.float32) + b1_ref[...]
    x = jnp.maximum(x, 0.0)
    x = jnp.dot(x, w2_ref[...], preferred_element_type=jnp.float32) + b2_ref[...]
    x = jnp.maximum(x, 0.0)
    x = jnp.dot(x, w3_ref[...], preferred_element_type=jnp.float32) + b3_ref[...]
    mu = jnp.mean(x, axis=-1, keepdims=True)
    xc = x - mu
    var = jnp.mean(xc * xc, axis=-1, keepdims=True)
    out_ref[...] = xc * lax.rsqrt(var + 1e-5) * gg_ref[...] + bb_ref[...]


def _compute_msg(g, ea, wc, b1, w2, b2, w3, b3, gg, bb):
    be = 2000
    nb = EP // be
    wspec = pl.BlockSpec((D, D), lambda i: (0, 0))
    vspec = pl.BlockSpec((1, D), lambda i: (0, 0))
    return pl.pallas_call(
        _msg_body,
        grid=(nb,),
        in_specs=[
            pl.BlockSpec((be, D // 2), lambda i: (i, 0)),
            pl.BlockSpec((be, EA), lambda i: (i, 0)),
            pl.BlockSpec((EA, D), lambda i: (0, 0)),
            vspec, wspec, vspec, wspec, vspec, vspec, vspec,
        ],
        out_specs=pl.BlockSpec((be, D), lambda i: (i, 0)),
        out_shape=jax.ShapeDtypeStruct((EP, D), jnp.float32),
    )(g, ea, wc, b1, w2, b2, w3, b3, gg, bb)


@functools.partial(
    pl.kernel,
    out_type=jax.ShapeDtypeStruct((NC, N, D), jnp.float32),
    mesh=_MESH,
    scratch_types=[
        pltpu.VMEM_SHARED((N, D), jnp.float32),
        pltpu.VMEM((NCH, CHUNK), jnp.int32),
        pltpu.VMEM((NBUF, CHUNK, D), jnp.float32),
    ] + [pltpu.SemaphoreType.DMA] * (2 * NBUF),
)
def _scatter_add(msg_hbm, dst_hbm, zeros_hbm, agg_hbm, acc, didx, mbuf,
                 *sems):
    c = lax.axis_index("c")
    s = lax.axis_index("s")
    wid = s * NC + c
    pltpu.sync_copy(zeros_hbm.at[pl.ds(0, ROWS_PT)],
                    acc.at[pl.ds(s * ROWS_PT, ROWS_PT)])

    @pl.when(s == NS - 1)
    def _zero_tail():
        pltpu.sync_copy(zeros_hbm.at[pl.ds(0, ROWS_TAIL)],
                        acc.at[pl.ds(NS * ROWS_PT, ROWS_TAIL)])

    plsc.subcore_barrier()
    pltpu.sync_copy(dst_hbm.at[wid], didx)
    base = wid * EPW
    semm = sems[0:NBUF]
    sems_ = sems[NBUF:2 * NBUF]

    def startl(j, b):
        pltpu.async_copy(msg_hbm.at[pl.ds(base + j * CHUNK, CHUNK)],
                         mbuf.at[b], semm[b])

    def waitl(b):
        pltpu.make_async_copy(msg_hbm.at[pl.ds(base, CHUNK)],
                              mbuf.at[b], semm[b]).wait()

    def starts(j, b):
        pltpu.async_copy(mbuf.at[b], acc.at[didx.at[j]], sems_[b], add=True)

    def waits(j, b):
        pltpu.make_async_copy(mbuf.at[b], acc.at[didx.at[j]],
                              sems_[b]).wait()

    for b in range(NBUF):
        startl(b, b)

    @pl.loop(0, NCH - 1, step=NBUF)
    def _grp(j):
        for b in range(NBUF):
            waitl(b)
            starts(j + b, b)
        for b in range(NBUF):
            jj = j + NBUF + b

            @pl.when(jj < NCH)
            def _pf(jj=jj, b=b):
                waits(jj - NBUF, b)
                startl(jj, b)

    waitl(0)
    starts(NCH - 1, 0)
    waits(NCH - 1, 0)
    for b in range(1, NBUF):
        waits(NCH - 1 - NBUF + b, b)

    plsc.subcore_barrier()
    pltpu.sync_copy(acc.at[pl.ds(s * ROWS_PT, ROWS_PT)],
                    agg_hbm.at[c, pl.ds(s * ROWS_PT, ROWS_PT)])

    @pl.when(s == NS - 1)
    def _drain_tail():
        pltpu.sync_copy(acc.at[pl.ds(NS * ROWS_PT, ROWS_TAIL)],
                        agg_hbm.at[c, pl.ds(NS * ROWS_PT, ROWS_TAIL)])


def _upd_body(h_ref, a0_ref, a1_ref, a2_ref, a3_ref, wa_ref, wb_ref, b1_ref,
              w2_ref, b2_ref, w3_ref, b3_ref, gg_ref, bb_ref, out_ref):
    hb = h_ref[...]
    agg = (a0_ref[...] + a1_ref[...]) + (a2_ref[...] + a3_ref[...])
    x = (jnp.dot(hb, wa_ref[...], preferred_element_type=jnp.float32)
         + jnp.dot(agg, wb_ref[...], preferred_element_type=jnp.float32)
         + b1_ref[...])
    x = jnp.maximum(x, 0.0)
    x = jnp.dot(x, w2_ref[...], preferred_element_type=jnp.float32) + b2_ref[...]
    x = jnp.maximum(x, 0.0)
    x = jnp.dot(x, w3_ref[...], preferred_element_type=jnp.float32) + b3_ref[...]
    mu = jnp.mean(x, axis=-1, keepdims=True)
    xc = x - mu
    var = jnp.mean(xc * xc, axis=-1, keepdims=True)
    out_ref[...] = xc * lax.rsqrt(var + 1e-5) * gg_ref[...] + bb_ref[...] + hb


def _compute_update(h, aggs, wa, wb, b1, w2, b2, w3, b3, gg, bb):
    nb = 10
    blk = N // nb
    wspec = pl.BlockSpec((D, D), lambda i: (0, 0))
    vspec = pl.BlockSpec((1, D), lambda i: (0, 0))
    rspec = pl.BlockSpec((blk, D), lambda i: (i, 0))
    return pl.pallas_call(
        _upd_body,
        grid=(nb,),
        in_specs=[rspec, rspec, rspec, rspec, rspec, wspec, wspec, vspec,
                  wspec, vspec, wspec, vspec, vspec, vspec],
        out_specs=rspec,
        out_shape=jax.ShapeDtypeStruct((N, D), jnp.float32),
    )(h, *aggs, wa, wb, b1, w2, b2, w3, b3, gg, bb)


def kernel(h, edge_index, edge_attr, mW1, mb1, mW2, mb2, mW3, mb3, mg, mbeta,
           uW1, ub1, uW2, ub2, uW3, ub3, ug, ubeta):
    src = edge_index[0].astype(jnp.int32)
    dst = edge_index[1].astype(jnp.int32)

    mA = mW1[:D]
    mB = mW1[D:2 * D]
    mC = mW1[2 * D:]
    uA = uW1[:D]
    uB = uW1[D:]

    mb1r = mb1.reshape(1, D)
    mb2r = mb2.reshape(1, D)
    mb3r = mb3.reshape(1, D)
    mgr = mg.reshape(1, D)
    mbetar = mbeta.reshape(1, D)
    ub1r = ub1.reshape(1, D)
    ub2r = ub2.reshape(1, D)
    ub3r = ub3.reshape(1, D)
    ugr = ug.reshape(1, D)
    ubetar = ubeta.reshape(1, D)

    t = _compute_pq(h, mA, mB)
    zeros = jnp.zeros((ROWS_PT, D), dtype=jnp.float32)

    src3ds, dst3ds, gs, msgs, aggs = [], [], [], [], []
    for part in range(NPART):
        sl = slice(part * EP, (part + 1) * EP)
        src3ds.append(src[sl].reshape(NW, NCH, CHUNK))
        dst3ds.append(dst[sl].reshape(NW, NCH, CHUNK))
        gs.append(_gather_add(t, src3ds[part], dst3ds[part]))
    for part in range(NPART):
        sl = slice(part * EP, (part + 1) * EP)
        msgs.append(_compute_msg(gs[part], edge_attr[sl], mC, mb1r, mW2,
                                 mb2r, mW3, mb3r, mgr, mbetar))
    for part in range(NPART):
        aggp = _scatter_add(msgs[part], dst3ds[part], zeros)
        aggs.append(aggp[0])
        aggs.append(aggp[1])

    h_new = _compute_update(h, aggs, uA, uB, ub1r, uW2, ub2r,
                            uW3, ub3r, ugr, ubetar)
    msg_full = jnp.concatenate(msgs, axis=0)
    return (h_new, msg_full)


# speedup vs baseline: 1.0333x; 1.0333x over previous
import functools

import jax
import jax.numpy as jnp
from jax import lax
from jax.experimental import pallas as pl
from jax.experimental.pallas import tpu as pltpu
from jax.experimental.pallas import tpu_sc as plsc

N = 10000
E = 320000
D = 128
EA = 16

NC, NS, L = 2, 16, 16
NW = NC * NS
EP = E
EPW = EP // NW
CHUNK = 40
NCH = EPW // CHUNK
ROWS_PT = 624
ROWS_TAIL = N - NS * ROWS_PT

_MESH = plsc.VectorSubcoreMesh(core_axis_name="c", subcore_axis_name="s")


def _rnd_bf16_bits(x):
    u = lax.bitcast_convert_type(x, jnp.uint32)
    return (u + jnp.uint32(0x7FFF) + ((u >> 16) & jnp.uint32(1))) >> 16


def _pq_body(h_ref, wa_ref, wb_ref, t_ref):
    hb = h_ref[...]
    pf = jnp.dot(hb, wa_ref[...], preferred_element_type=jnp.float32)
    qf = jnp.dot(hb, wb_ref[...], preferred_element_type=jnp.float32)
    packed = _rnd_bf16_bits(pf) | (_rnd_bf16_bits(qf) << 16)
    t_ref[...] = lax.bitcast_convert_type(packed, jnp.int32)


def _compute_pq(h, wa, wb):
    nb = 10
    blk = N // nb
    return pl.pallas_call(
        _pq_body,
        grid=(nb,),
        in_specs=[
            pl.BlockSpec((blk, D), lambda i: (i, 0)),
            pl.BlockSpec((D, D), lambda i: (0, 0)),
            pl.BlockSpec((D, D), lambda i: (0, 0)),
        ],
        out_specs=pl.BlockSpec((blk, D), lambda i: (i, 0)),
        out_shape=jax.ShapeDtypeStruct((N, D), jnp.int32),
    )(h, wa, wb)


NBUF = 3
GNBUF = 2
assert NCH % 4 == 2


@functools.partial(
    pl.kernel,
    out_type=jax.ShapeDtypeStruct((EP, D // 2), jnp.int32),
    mesh=_MESH,
    scratch_types=[
        pltpu.VMEM_SHARED((N, D), jnp.int32),
        pltpu.VMEM((4, CHUNK), jnp.int32),
        pltpu.VMEM((4, CHUNK), jnp.int32),
        pltpu.VMEM((GNBUF, CHUNK, D), jnp.int32),
        pltpu.VMEM((GNBUF, CHUNK, D), jnp.int32),
        pltpu.VMEM((GNBUF, CHUNK, D // 2), jnp.int32),
    ] + [pltpu.SemaphoreType.DMA] * (3 * GNBUF + 8),
)
def _gather_add(t_hbm, src_hbm, dst_hbm, g_hbm,
                spt, sidx, didx, pbuf, qbuf, gbuf, *sems):
    c = lax.axis_index("c")
    s = lax.axis_index("s")
    wid = s * NC + c
    base = wid * EPW

    pltpu.sync_copy(t_hbm.at[pl.ds(s * ROWS_PT, ROWS_PT)],
                    spt.at[pl.ds(s * ROWS_PT, ROWS_PT)])

    @pl.when(s == NS - 1)
    def _stage_tail():
        pltpu.sync_copy(t_hbm.at[pl.ds(NS * ROWS_PT, ROWS_TAIL)],
                        spt.at[pl.ds(NS * ROWS_PT, ROWS_TAIL)])

    plsc.subcore_barrier()

    semp = sems[0:GNBUF]
    semq = sems[GNBUF:2 * GNBUF]
    semw = sems[2 * GNBUF:3 * GNBUF]
    semi = sems[3 * GNBUF:3 * GNBUF + 4]
    semj = sems[3 * GNBUF + 4:3 * GNBUF + 8]

    def startidx(j, i4):
        pltpu.async_copy(src_hbm.at[wid, j], sidx.at[i4], semi[i4])
        pltpu.async_copy(dst_hbm.at[wid, j], didx.at[i4], semj[i4])

    def waitidx(j, i4):
        pltpu.make_async_copy(src_hbm.at[wid, j], sidx.at[i4],
                              semi[i4]).wait()
        pltpu.make_async_copy(dst_hbm.at[wid, j], didx.at[i4],
                              semj[i4]).wait()

    def startg(b, i4):
        pltpu.async_copy(spt.at[sidx.at[i4]], pbuf.at[b], semp[b])
        pltpu.async_copy(spt.at[didx.at[i4]], qbuf.at[b], semq[b])

    def waitg(b, i4):
        pltpu.make_async_copy(spt.at[sidx.at[i4]], pbuf.at[b],
                              semp[b]).wait()
        pltpu.make_async_copy(spt.at[didx.at[i4]], qbuf.at[b],
                              semq[b]).wait()

    def startw(j, b):
        pltpu.async_copy(gbuf.at[b],
                         g_hbm.at[pl.ds(base + j * CHUNK, CHUNK)], semw[b])

    def waitw(b):
        pltpu.make_async_copy(gbuf.at[b],
                              g_hbm.at[pl.ds(base, CHUNK)], semw[b]).wait()

    kone = jnp.full((L,), 1, jnp.int32)
    khalf = jnp.full((L,), 0x7FFF, jnp.int32)
    kmhi = jnp.full((L,), -65536, jnp.int32)
    k16 = jnp.full((L,), 16, jnp.int32)

    def _to_bf16_bits(sf):
        u = lax.bitcast_convert_type(sf, jnp.int32)
        r = u + khalf + (lax.shift_right_logical(u, k16) & kone)
        return lax.shift_right_logical(r, k16)

    def adds(j, b):
        @pl.loop(0, CHUNK)
        def _row(r):
            for v in range(D // 2 // L):
                lo = pl.ds(v * L, L)
                hi = pl.ds(D // 2 + v * L, L)
                p_lo = lax.bitcast_convert_type(
                    lax.shift_left(pbuf[b, r, lo], k16), jnp.float32)
                q_lo = lax.bitcast_convert_type(
                    qbuf[b, r, lo] & kmhi, jnp.float32)
                p_hi = lax.bitcast_convert_type(
                    lax.shift_left(pbuf[b, r, hi], k16), jnp.float32)
                q_hi = lax.bitcast_convert_type(
                    qbuf[b, r, hi] & kmhi, jnp.float32)
                r_lo = _to_bf16_bits(p_lo + q_lo)
                r_hi = _to_bf16_bits(p_hi + q_hi)
                gbuf[b, r, lo] = r_lo | lax.shift_left(r_hi, k16)

    for k in range(4):
        startidx(k, k)
    waitidx(0, 0)
    startg(0, 0)
    waitidx(1, 1)
    startg(1, 1)

    @pl.loop(0, NCH - 2, step=4)
    def _grp(j):
        for k in range(4):
            c = j + k
            b = k % 2
            waitg(b, k)

            @pl.when(c >= 2)
            def _ww(b=b):
                waitw(b)

            adds(c, b)
            startw(c, b)
            cn = c + 2

            @pl.when(cn < NCH)
            def _pf(cn=cn, b=b, i4=(k + 2) % 4):
                waitidx(cn, i4)
                startg(b, i4)

            ci = c + 4

            @pl.when(ci < NCH)
            def _pfi(ci=ci, i4=k):
                startidx(ci, i4)

    for c in (NCH - 2, NCH - 1):
        b = c % 2
        waitg(b, c % 4)
        waitw(b)
        adds(c, b)
        startw(c, b)
    for b in range(GNBUF):
        waitw(b)


def _unpack_bf16(w):
    u = lax.bitcast_convert_type(w, jnp.uint32)
    lo = lax.bitcast_convert_type(u << 16, jnp.float32)
    hi = lax.bitcast_convert_type(u & jnp.uint32(0xFFFF0000), jnp.float32)
    return jnp.concatenate([lo, hi], axis=1)


def _msg_body(g_ref, ea_ref, wc_ref, b1_ref, w2_ref, b2_ref, w3_ref, b3_ref,
              gg_ref, bb_ref, out_ref):
    x = _unpack_bf16(g_ref[...]) + jnp.dot(
        ea_ref[...], wc_ref[...],
        preferred_element_type=jnp.float32) + b1_ref[...]
    x = jnp.maximum(x, 0.0)
    x = jnp.dot(x, w2_ref[...], preferred_element_type=jnp.float32) + b2_ref[...]
    x = jnp.maximum(x, 0.0)
    x = jnp.dot(x, w3_ref[...], preferred_element_type=jnp.float32) + b3_ref[...]
    mu = jnp.mean(x, axis=-1, keepdims=True)
    xc = x - mu
    var = jnp.mean(xc * xc, axis=-1, keepdims=True)
    out_ref[...] = xc * lax.rsqrt(var + 1e-5) * gg_ref[...] + bb_ref[...]


def _compute_msg(g, ea, wc, b1, w2, b2, w3, b3, gg, bb):
    be = 2000
    nb = EP // be
    wspec = pl.BlockSpec((D, D), lambda i: (0, 0))
    vspec = pl.BlockSpec((1, D), lambda i: (0, 0))
    return pl.pallas_call(
        _msg_body,
        grid=(nb,),
        in_specs=[
            pl.BlockSpec((be, D // 2), lambda i: (i, 0)),
            pl.BlockSpec((be, EA), lambda i: (i, 0)),
            pl.BlockSpec((EA, D), lambda i: (0, 0)),
            vspec, wspec, vspec, wspec, vspec, vspec, vspec,
        ],
        out_specs=pl.BlockSpec((be, D), lambda i: (i, 0)),
        out_shape=jax.ShapeDtypeStruct((EP, D), jnp.float32),
    )(g, ea, wc, b1, w2, b2, w3, b3, gg, bb)


@functools.partial(
    pl.kernel,
    out_type=jax.ShapeDtypeStruct((NC, N, D), jnp.float32),
    mesh=_MESH,
    scratch_types=[
        pltpu.VMEM_SHARED((N, D), jnp.float32),
        pltpu.VMEM((NCH, CHUNK), jnp.int32),
        pltpu.VMEM((NBUF, CHUNK, D), jnp.float32),
    ] + [pltpu.SemaphoreType.DMA] * (2 * NBUF),
)
def _scatter_add(msg_hbm, dst_hbm, zeros_hbm, agg_hbm, acc, didx, mbuf,
                 *sems):
    c = lax.axis_index("c")
    s = lax.axis_index("s")
    wid = s * NC + c
    pltpu.sync_copy(zeros_hbm.at[pl.ds(0, ROWS_PT)],
                    acc.at[pl.ds(s * ROWS_PT, ROWS_PT)])

    @pl.when(s == NS - 1)
    def _zero_tail():
        pltpu.sync_copy(zeros_hbm.at[pl.ds(0, ROWS_TAIL)],
                        acc.at[pl.ds(NS * ROWS_PT, ROWS_TAIL)])

    plsc.subcore_barrier()
    pltpu.sync_copy(dst_hbm.at[wid], didx)
    base = wid * EPW
    semm = sems[0:NBUF]
    sems_ = sems[NBUF:2 * NBUF]

    def startl(j, b):
        pltpu.async_copy(msg_hbm.at[pl.ds(base + j * CHUNK, CHUNK)],
                         mbuf.at[b], semm[b])

    def waitl(b):
        pltpu.make_async_copy(msg_hbm.at[pl.ds(base, CHUNK)],
                              mbuf.at[b], semm[b]).wait()

    def starts(j, b):
        pltpu.async_copy(mbuf.at[b], acc.at[didx.at[j]], sems_[b], add=True)

    def waits(j, b):
        pltpu.make_async_copy(mbuf.at[b], acc.at[didx.at[j]],
                              sems_[b]).wait()

    for b in range(NBUF):
        startl(b, b)

    @pl.loop(0, NCH - 2, step=NBUF)
    def _grp(j):
        for b in range(NBUF):
            waitl(b)
            starts(j + b, b)
        for b in range(NBUF):
            jj = j + NBUF + b

            @pl.when(jj < NCH)
            def _pf(jj=jj, b=b):
                waits(jj - NBUF, b)
                startl(jj, b)

    for ct in range(NCH - NCH % NBUF, NCH):
        b = ct % NBUF
        waitl(b)
        starts(ct, b)
    for b in range(NBUF):
        waits(NCH - NBUF + ((b - NCH) % NBUF), b)

    plsc.subcore_barrier()
    pltpu.sync_copy(acc.at[pl.ds(s * ROWS_PT, ROWS_PT)],
                    agg_hbm.at[c, pl.ds(s * ROWS_PT, ROWS_PT)])

    @pl.when(s == NS - 1)
    def _drain_tail():
        pltpu.sync_copy(acc.at[pl.ds(NS * ROWS_PT, ROWS_TAIL)],
                        agg_hbm.at[c, pl.ds(NS * ROWS_PT, ROWS_TAIL)])


def _upd_body(h_ref, a0_ref, a1_ref, wa_ref, wb_ref, b1_ref,
              w2_ref, b2_ref, w3_ref, b3_ref, gg_ref, bb_ref, out_ref):
    hb = h_ref[...]
    agg = a0_ref[...] + a1_ref[...]
    x = (jnp.dot(hb, wa_ref[...], preferred_element_type=jnp.float32)
         + jnp.dot(agg, wb_ref[...], preferred_element_type=jnp.float32)
         + b1_ref[...])
    x = jnp.maximum(x, 0.0)
    x = jnp.dot(x, w2_ref[...], preferred_element_type=jnp.float32) + b2_ref[...]
    x = jnp.maximum(x, 0.0)
    x = jnp.dot(x, w3_ref[...], preferred_element_type=jnp.float32) + b3_ref[...]
    mu = jnp.mean(x, axis=-1, keepdims=True)
    xc = x - mu
    var = jnp.mean(xc * xc, axis=-1, keepdims=True)
    out_ref[...] = xc * lax.rsqrt(var + 1e-5) * gg_ref[...] + bb_ref[...] + hb


def _compute_update(h, aggs, wa, wb, b1, w2, b2, w3, b3, gg, bb):
    nb = 10
    blk = N // nb
    wspec = pl.BlockSpec((D, D), lambda i: (0, 0))
    vspec = pl.BlockSpec((1, D), lambda i: (0, 0))
    rspec = pl.BlockSpec((blk, D), lambda i: (i, 0))
    return pl.pallas_call(
        _upd_body,
        grid=(nb,),
        in_specs=[rspec, rspec, rspec, wspec, wspec, vspec,
                  wspec, vspec, wspec, vspec, vspec, vspec],
        out_specs=rspec,
        out_shape=jax.ShapeDtypeStruct((N, D), jnp.float32),
    )(h, *aggs, wa, wb, b1, w2, b2, w3, b3, gg, bb)


def kernel(h, edge_index, edge_attr, mW1, mb1, mW2, mb2, mW3, mb3, mg, mbeta,
           uW1, ub1, uW2, ub2, uW3, ub3, ug, ubeta):
    src = edge_index[0].astype(jnp.int32)
    dst = edge_index[1].astype(jnp.int32)

    mA = mW1[:D]
    mB = mW1[D:2 * D]
    mC = mW1[2 * D:]
    uA = uW1[:D]
    uB = uW1[D:]

    mb1r = mb1.reshape(1, D)
    mb2r = mb2.reshape(1, D)
    mb3r = mb3.reshape(1, D)
    mgr = mg.reshape(1, D)
    mbetar = mbeta.reshape(1, D)
    ub1r = ub1.reshape(1, D)
    ub2r = ub2.reshape(1, D)
    ub3r = ub3.reshape(1, D)
    ugr = ug.reshape(1, D)
    ubetar = ubeta.reshape(1, D)

    t = _compute_pq(h, mA, mB)
    zeros = jnp.zeros((ROWS_PT, D), dtype=jnp.float32)

    src3d = src.reshape(NW, NCH, CHUNK)
    dst3d = dst.reshape(NW, NCH, CHUNK)
    g = _gather_add(t, src3d, dst3d)
    msg = _compute_msg(g, edge_attr, mC, mb1r, mW2, mb2r, mW3, mb3r,
                       mgr, mbetar)
    aggp = _scatter_add(msg, dst3d, zeros)
    h_new = _compute_update(h, [aggp[0], aggp[1]], uA, uB, ub1r, uW2, ub2r,
                            uW3, ub3r, ugr, ubetar)
    return (h_new, msg)


# speedup vs baseline: 1.1231x; 1.0869x over previous
import functools

import jax
import jax.numpy as jnp
from jax import lax
from jax.experimental import pallas as pl
from jax.experimental.pallas import tpu as pltpu
from jax.experimental.pallas import tpu_sc as plsc

N = 10000
E = 320000
D = 128
EA = 16

NC, NS, L = 2, 16, 16
NW = NC * NS
EP = E
EPW = EP // NW
CHUNK = 40
NCH = EPW // CHUNK
ROWS_PT = 624
ROWS_TAIL = N - NS * ROWS_PT

_MESH = plsc.VectorSubcoreMesh(core_axis_name="c", subcore_axis_name="s")


def _rnd_bf16_bits(x):
    u = lax.bitcast_convert_type(x, jnp.uint32)
    return (u + jnp.uint32(0x7FFF) + ((u >> 16) & jnp.uint32(1))) >> 16


def _pq_body(h_ref, wa_ref, wb_ref, t_ref):
    hb = h_ref[...]
    pf = jnp.dot(hb, wa_ref[...], preferred_element_type=jnp.float32)
    qf = jnp.dot(hb, wb_ref[...], preferred_element_type=jnp.float32)
    packed = _rnd_bf16_bits(pf) | (_rnd_bf16_bits(qf) << 16)
    t_ref[...] = lax.bitcast_convert_type(packed, jnp.int32)


def _compute_pq(h, wa, wb):
    nb = 10
    blk = N // nb
    return pl.pallas_call(
        _pq_body,
        grid=(nb,),
        in_specs=[
            pl.BlockSpec((blk, D), lambda i: (i, 0)),
            pl.BlockSpec((D, D), lambda i: (0, 0)),
            pl.BlockSpec((D, D), lambda i: (0, 0)),
        ],
        out_specs=pl.BlockSpec((blk, D), lambda i: (i, 0)),
        out_shape=jax.ShapeDtypeStruct((N, D), jnp.int32),
    )(h, wa, wb)


NBUF = 3
GNBUF = 2
assert NCH % 4 == 2


@functools.partial(
    pl.kernel,
    out_type=jax.ShapeDtypeStruct((EP, D // 2), jnp.int32),
    mesh=_MESH,
    scratch_types=[
        pltpu.VMEM_SHARED((N, D), jnp.int32),
        pltpu.VMEM((4, CHUNK), jnp.int32),
        pltpu.VMEM((4, CHUNK), jnp.int32),
        pltpu.VMEM((GNBUF, CHUNK, D), jnp.int32),
        pltpu.VMEM((GNBUF, CHUNK, D), jnp.int32),
        pltpu.VMEM((GNBUF, CHUNK, D // 2), jnp.int32),
    ] + [pltpu.SemaphoreType.DMA] * (3 * GNBUF + 8),
)
def _gather_add(t_hbm, src_hbm, dst_hbm, g_hbm,
                spt, sidx, didx, pbuf, qbuf, gbuf, *sems):
    c = lax.axis_index("c")
    s = lax.axis_index("s")
    wid = s * NC + c
    base = wid * EPW

    pltpu.sync_copy(t_hbm.at[pl.ds(s * ROWS_PT, ROWS_PT)],
                    spt.at[pl.ds(s * ROWS_PT, ROWS_PT)])

    @pl.when(s == NS - 1)
    def _stage_tail():
        pltpu.sync_copy(t_hbm.at[pl.ds(NS * ROWS_PT, ROWS_TAIL)],
                        spt.at[pl.ds(NS * ROWS_PT, ROWS_TAIL)])

    plsc.subcore_barrier()

    semp = sems[0:GNBUF]
    semq = sems[GNBUF:2 * GNBUF]
    semw = sems[2 * GNBUF:3 * GNBUF]
    semi = sems[3 * GNBUF:3 * GNBUF + 4]
    semj = sems[3 * GNBUF + 4:3 * GNBUF + 8]

    def startidx(j, i4):
        pltpu.async_copy(src_hbm.at[wid, j], sidx.at[i4], semi[i4])
        pltpu.async_copy(dst_hbm.at[wid, j], didx.at[i4], semj[i4])

    def waitidx(j, i4):
        pltpu.make_async_copy(src_hbm.at[wid, j], sidx.at[i4],
                              semi[i4]).wait()
        pltpu.make_async_copy(dst_hbm.at[wid, j], didx.at[i4],
                              semj[i4]).wait()

    def startg(b, i4):
        pltpu.async_copy(spt.at[sidx.at[i4]], pbuf.at[b], semp[b])
        pltpu.async_copy(spt.at[didx.at[i4]], qbuf.at[b], semq[b])

    def waitg(b, i4):
        pltpu.make_async_copy(spt.at[sidx.at[i4]], pbuf.at[b],
                              semp[b]).wait()
        pltpu.make_async_copy(spt.at[didx.at[i4]], qbuf.at[b],
                              semq[b]).wait()

    def startw(j, b):
        pltpu.async_copy(gbuf.at[b],
                         g_hbm.at[pl.ds(base + j * CHUNK, CHUNK)], semw[b])

    def waitw(b):
        pltpu.make_async_copy(gbuf.at[b],
                              g_hbm.at[pl.ds(base, CHUNK)], semw[b]).wait()

    khalf = jnp.full((L,), 0x8000, jnp.int32)
    kmhi = jnp.full((L,), -65536, jnp.int32)
    k16 = jnp.full((L,), 16, jnp.int32)

    def _to_bf16_bits(sf):
        u = lax.bitcast_convert_type(sf, jnp.int32)
        return lax.shift_right_logical(u + khalf, k16)

    def adds(j, b):
        @pl.loop(0, CHUNK)
        def _row(r):
            for v in range(D // 2 // L):
                lo = pl.ds(v * L, L)
                hi = pl.ds(D // 2 + v * L, L)
                p_lo = lax.bitcast_convert_type(
                    lax.shift_left(pbuf[b, r, lo], k16), jnp.float32)
                q_lo = lax.bitcast_convert_type(
                    qbuf[b, r, lo] & kmhi, jnp.float32)
                p_hi = lax.bitcast_convert_type(
                    lax.shift_left(pbuf[b, r, hi], k16), jnp.float32)
                q_hi = lax.bitcast_convert_type(
                    qbuf[b, r, hi] & kmhi, jnp.float32)
                r_lo = _to_bf16_bits(p_lo + q_lo)
                r_hi = _to_bf16_bits(p_hi + q_hi)
                gbuf[b, r, lo] = r_lo | lax.shift_left(r_hi, k16)

    for k in range(4):
        startidx(k, k)
    waitidx(0, 0)
    startg(0, 0)
    waitidx(1, 1)
    startg(1, 1)

    @pl.loop(0, NCH - 2, step=4)
    def _grp(j):
        for k in range(4):
            c = j + k
            b = k % 2
            waitg(b, k)

            @pl.when(c >= 2)
            def _ww(b=b):
                waitw(b)

            adds(c, b)
            startw(c, b)
            cn = c + 2

            @pl.when(cn < NCH)
            def _pf(cn=cn, b=b, i4=(k + 2) % 4):
                waitidx(cn, i4)
                startg(b, i4)

            ci = c + 4

            @pl.when(ci < NCH)
            def _pfi(ci=ci, i4=k):
                startidx(ci, i4)

    for c in (NCH - 2, NCH - 1):
        b = c % 2
        waitg(b, c % 4)
        waitw(b)
        adds(c, b)
        startw(c, b)
    for b in range(GNBUF):
        waitw(b)


def _unpack_bf16(w):
    u = lax.bitcast_convert_type(w, jnp.uint32)
    lo = lax.bitcast_convert_type(u << 16, jnp.float32)
    hi = lax.bitcast_convert_type(u & jnp.uint32(0xFFFF0000), jnp.float32)
    return jnp.concatenate([lo, hi], axis=1)


def _msg_body(g_ref, ea_ref, wc_ref, b1_ref, w2_ref, b2_ref, w3_ref, b3_ref,
              gg_ref, bb_ref, out_ref):
    x = _unpack_bf16(g_ref[...]) + jnp.dot(
        ea_ref[...], wc_ref[...],
        preferred_element_type=jnp.float32) + b1_ref[...]
    x = jnp.maximum(x, 0.0)
    x = jnp.dot(x, w2_ref[...], preferred_element_type=jnp.float32) + b2_ref[...]
    x = jnp.maximum(x, 0.0)
    x = jnp.dot(x, w3_ref[...], preferred_element_type=jnp.float32) + b3_ref[...]
    mu = jnp.mean(x, axis=-1, keepdims=True)
    xc = x - mu
    var = jnp.mean(xc * xc, axis=-1, keepdims=True)
    out_ref[...] = xc * lax.rsqrt(var + 1e-5) * gg_ref[...] + bb_ref[...]


def _compute_msg(g, ea, wc, b1, w2, b2, w3, b3, gg, bb):
    be = 3200
    nb = EP // be
    wspec = pl.BlockSpec((D, D), lambda i: (0, 0))
    vspec = pl.BlockSpec((1, D), lambda i: (0, 0))
    return pl.pallas_call(
        _msg_body,
        grid=(nb,),
        in_specs=[
            pl.BlockSpec((be, D // 2), lambda i: (i, 0)),
            pl.BlockSpec((be, EA), lambda i: (i, 0)),
            pl.BlockSpec((EA, D), lambda i: (0, 0)),
            vspec, wspec, vspec, wspec, vspec, vspec, vspec,
        ],
        out_specs=pl.BlockSpec((be, D), lambda i: (i, 0)),
        out_shape=jax.ShapeDtypeStruct((EP, D), jnp.float32),
    )(g, ea, wc, b1, w2, b2, w3, b3, gg, bb)


@functools.partial(
    pl.kernel,
    out_type=jax.ShapeDtypeStruct((NC, N, D), jnp.float32),
    mesh=_MESH,
    scratch_types=[
        pltpu.VMEM_SHARED((N, D), jnp.float32),
        pltpu.VMEM((NCH, CHUNK), jnp.int32),
        pltpu.VMEM((NBUF, CHUNK, D), jnp.float32),
    ] + [pltpu.SemaphoreType.DMA] * (2 * NBUF),
)
def _scatter_add(msg_hbm, dst_hbm, zeros_hbm, agg_hbm, acc, didx, mbuf,
                 *sems):
    c = lax.axis_index("c")
    s = lax.axis_index("s")
    wid = s * NC + c
    pltpu.sync_copy(zeros_hbm.at[pl.ds(0, ROWS_PT)],
                    acc.at[pl.ds(s * ROWS_PT, ROWS_PT)])

    @pl.when(s == NS - 1)
    def _zero_tail():
        pltpu.sync_copy(zeros_hbm.at[pl.ds(0, ROWS_TAIL)],
                        acc.at[pl.ds(NS * ROWS_PT, ROWS_TAIL)])

    plsc.subcore_barrier()
    pltpu.sync_copy(dst_hbm.at[wid], didx)
    base = wid * EPW
    semm = sems[0:NBUF]
    sems_ = sems[NBUF:2 * NBUF]

    def startl(j, b):
        pltpu.async_copy(msg_hbm.at[pl.ds(base + j * CHUNK, CHUNK)],
                         mbuf.at[b], semm[b])

    def waitl(b):
        pltpu.make_async_copy(msg_hbm.at[pl.ds(base, CHUNK)],
                              mbuf.at[b], semm[b]).wait()

    def starts(j, b):
        pltpu.async_copy(mbuf.at[b], acc.at[didx.at[j]], sems_[b], add=True)

    def waits(j, b):
        pltpu.make_async_copy(mbuf.at[b], acc.at[didx.at[j]],
                              sems_[b]).wait()

    for b in range(NBUF):
        startl(b, b)

    @pl.loop(0, NCH - 2, step=NBUF)
    def _grp(j):
        for b in range(NBUF):
            waitl(b)
            starts(j + b, b)
        for b in range(NBUF):
            jj = j + NBUF + b

            @pl.when(jj < NCH)
            def _pf(jj=jj, b=b):
                waits(jj - NBUF, b)
                startl(jj, b)

    for ct in range(NCH - NCH % NBUF, NCH):
        b = ct % NBUF
        waitl(b)
        starts(ct, b)
    for b in range(NBUF):
        waits(NCH - NBUF + ((b - NCH) % NBUF), b)

    plsc.subcore_barrier()
    pltpu.sync_copy(acc.at[pl.ds(s * ROWS_PT, ROWS_PT)],
                    agg_hbm.at[c, pl.ds(s * ROWS_PT, ROWS_PT)])

    @pl.when(s == NS - 1)
    def _drain_tail():
        pltpu.sync_copy(acc.at[pl.ds(NS * ROWS_PT, ROWS_TAIL)],
                        agg_hbm.at[c, pl.ds(NS * ROWS_PT, ROWS_TAIL)])


def _upd_body(h_ref, a0_ref, a1_ref, wa_ref, wb_ref, b1_ref,
              w2_ref, b2_ref, w3_ref, b3_ref, gg_ref, bb_ref, out_ref):
    hb = h_ref[...]
    agg = a0_ref[...] + a1_ref[...]
    x = (jnp.dot(hb, wa_ref[...], preferred_element_type=jnp.float32)
         + jnp.dot(agg, wb_ref[...], preferred_element_type=jnp.float32)
         + b1_ref[...])
    x = jnp.maximum(x, 0.0)
    x = jnp.dot(x, w2_ref[...], preferred_element_type=jnp.float32) + b2_ref[...]
    x = jnp.maximum(x, 0.0)
    x = jnp.dot(x, w3_ref[...], preferred_element_type=jnp.float32) + b3_ref[...]
    mu = jnp.mean(x, axis=-1, keepdims=True)
    xc = x - mu
    var = jnp.mean(xc * xc, axis=-1, keepdims=True)
    out_ref[...] = xc * lax.rsqrt(var + 1e-5) * gg_ref[...] + bb_ref[...] + hb


def _compute_update(h, aggs, wa, wb, b1, w2, b2, w3, b3, gg, bb):
    nb = 10
    blk = N // nb
    wspec = pl.BlockSpec((D, D), lambda i: (0, 0))
    vspec = pl.BlockSpec((1, D), lambda i: (0, 0))
    rspec = pl.BlockSpec((blk, D), lambda i: (i, 0))
    return pl.pallas_call(
        _upd_body,
        grid=(nb,),
        in_specs=[rspec, rspec, rspec, wspec, wspec, vspec,
                  wspec, vspec, wspec, vspec, vspec, vspec],
        out_specs=rspec,
        out_shape=jax.ShapeDtypeStruct((N, D), jnp.float32),
    )(h, *aggs, wa, wb, b1, w2, b2, w3, b3, gg, bb)


def kernel(h, edge_index, edge_attr, mW1, mb1, mW2, mb2, mW3, mb3, mg, mbeta,
           uW1, ub1, uW2, ub2, uW3, ub3, ug, ubeta):
    src = edge_index[0].astype(jnp.int32)
    dst = edge_index[1].astype(jnp.int32)

    mA = mW1[:D]
    mB = mW1[D:2 * D]
    mC = mW1[2 * D:]
    uA = uW1[:D]
    uB = uW1[D:]

    mb1r = mb1.reshape(1, D)
    mb2r = mb2.reshape(1, D)
    mb3r = mb3.reshape(1, D)
    mgr = mg.reshape(1, D)
    mbetar = mbeta.reshape(1, D)
    ub1r = ub1.reshape(1, D)
    ub2r = ub2.reshape(1, D)
    ub3r = ub3.reshape(1, D)
    ugr = ug.reshape(1, D)
    ubetar = ubeta.reshape(1, D)

    t = _compute_pq(h, mA, mB)
    zeros = jnp.zeros((ROWS_PT, D), dtype=jnp.float32)

    src3d = src.reshape(NW, NCH, CHUNK)
    dst3d = dst.reshape(NW, NCH, CHUNK)
    g = _gather_add(t, src3d, dst3d)
    msg = _compute_msg(g, edge_attr, mC, mb1r, mW2, mb2r, mW3, mb3r,
                       mgr, mbetar)
    aggp = _scatter_add(msg, dst3d, zeros)
    h_new = _compute_update(h, [aggp[0], aggp[1]], uA, uB, ub1r, uW2, ub2r,
                            uW3, ub3r, ugr, ubetar)
    return (h_new, msg)
